# Initial kernel scaffold; baseline (speedup 1.0000x reference)
import jax
import jax.numpy as jnp
from jax.experimental import pallas as pl


def kernel(geom_feats, x):
    raise NotImplementedError("write your pallas kernel here")


# speedup vs baseline: 2.3938x; 2.3938x over previous
import dataclasses
import functools

import jax
import jax.numpy as jnp
from jax import lax
from jax.experimental import pallas as pl
from jax.experimental.pallas import tpu as pltpu
from jax.experimental.pallas import tpu_sc as plsc

XB = (-51.2, 51.2, 0.8)
YB = (-51.2, 51.2, 0.8)
ZB = (-10.0, 10.0, 20.0)
NX = 128
NY = 128
NCELL = NX * NY

B = 6
D = 41
C = 64
H = 32
W = 88
HW = H * W
BD = B * D

CH_PER_TILE = 4
TILES_PER_BATCH = C // CH_PER_TILE
BATCHES_PER_PASS = 2
NUM_PASSES = B // BATCHES_PER_PASS
LANES = 16
GROUPS = HW // LANES


def _idx_body(g_ref, idx_ref):
    gx = g_ref[0]
    gy = g_ref[1]
    gz = g_ref[2]
    keep = (
        (gx >= XB[0]) & (gx < XB[1])
        & (gy >= YB[0]) & (gy < YB[1])
        & (gz >= ZB[0]) & (gz < ZB[1])
    )
    ix = jnp.clip(((gx - XB[0]) * (1.0 / XB[2])).astype(jnp.int32), 0, NX - 1)
    iy = jnp.clip(((gy - YB[0]) * (1.0 / YB[2])).astype(jnp.int32), 0, NY - 1)
    flat = ix * NY + iy
    idx_ref[...] = jnp.where(keep, flat, -1)


def _compute_idx(gt):
    return pl.pallas_call(
        _idx_body,
        out_shape=jax.ShapeDtypeStruct((BD, HW), jnp.int32),
    )(gt)


def _sc_scatter(x4, idx3):
    mesh = plsc.VectorSubcoreMesh(core_axis_name="c", subcore_axis_name="s")
    cp = pltpu.CompilerParams()
    if "needs_layout_passes" in pltpu.CompilerParams.__dataclass_fields__:
        cp = dataclasses.replace(cp, needs_layout_passes=False)

    @functools.partial(
        pl.kernel,
        compiler_params=cp,
        out_type=jax.ShapeDtypeStruct((B, C, NCELL), jnp.float32),
        mesh=mesh,
        scratch_types=(
            [pltpu.VMEM((NCELL,), jnp.float32) for _ in range(CH_PER_TILE)]
            + [
                pltpu.VMEM((CH_PER_TILE, HW), jnp.float32),
                pltpu.VMEM((HW,), jnp.int32),
            ]
        ),
    )
    def sc_kernel(x_hbm, idx_hbm, out_hbm, g0, g1, g2, g3, xbuf, ibuf):
        grids = [g0, g1, g2, g3]
        wid = lax.axis_index("s") * 2 + lax.axis_index("c")
        cgrp = lax.rem(wid, TILES_PER_BATCH)
        bsel = wid // TILES_PER_BATCH
        c0 = cgrp * CH_PER_TILE

        zero16 = jnp.zeros((LANES,), jnp.float32)

        for p in range(NUM_PASSES):
            b = p * BATCHES_PER_PASS + bsel

            @pl.loop(0, NCELL // LANES)
            def _zero(i):
                for c in range(CH_PER_TILE):
                    grids[c][pl.ds(i * LANES, LANES)] = zero16

            @pl.loop(0, D)
            def _per_d(d):
                pltpu.sync_copy(x_hbm.at[b, d, pl.ds(c0, CH_PER_TILE)], xbuf)
                pltpu.sync_copy(idx_hbm.at[b, d], ibuf)

                @pl.loop(0, GROUPS)
                def _per_group(g):
                    base = g * LANES
                    idxv = ibuf[pl.ds(base, LANES)]
                    mask = idxv >= 0
                    for c in range(CH_PER_TILE):
                        vals = xbuf[c, pl.ds(base, LANES)]
                        plsc.addupdate_scatter(grids[c], [idxv], vals, mask=mask)

            for c in range(CH_PER_TILE):
                pltpu.sync_copy(grids[c], out_hbm.at[b, c0 + c])

    return sc_kernel(x4, idx3)


@jax.jit
def kernel(geom_feats, x):
    gt = jnp.transpose(geom_feats.reshape(BD, HW, 3), (2, 0, 1))
    x4 = x.reshape(B, D, C, HW)
    idx = _compute_idx(gt)
    out = _sc_scatter(x4, idx.reshape(B, D, HW))
    return out.reshape(B, C, NX, NY)


# speedup vs baseline: 2.8771x; 1.2019x over previous
import dataclasses
import functools

import jax
import jax.numpy as jnp
from jax import lax
from jax.experimental import pallas as pl
from jax.experimental.pallas import tpu as pltpu
from jax.experimental.pallas import tpu_sc as plsc

XB = (-51.2, 51.2, 0.8)
YB = (-51.2, 51.2, 0.8)
ZB = (-10.0, 10.0, 20.0)
NX = 128
NY = 128
NCELL = NX * NY

B = 6
D = 41
C = 64
H = 32
W = 88
HW = H * W
BD = B * D

CH_PER_TILE = 4
TILES_PER_BATCH = C // CH_PER_TILE
BATCHES_PER_PASS = 2
NUM_PASSES = B // BATCHES_PER_PASS
LANES = 16
GROUPS = HW // LANES


def _idx_body(g_ref, idx_ref):
    gx = g_ref[0]
    gy = g_ref[1]
    gz = g_ref[2]
    keep = (
        (gx >= XB[0]) & (gx < XB[1])
        & (gy >= YB[0]) & (gy < YB[1])
        & (gz >= ZB[0]) & (gz < ZB[1])
    )
    ix = jnp.clip(((gx - XB[0]) * (1.0 / XB[2])).astype(jnp.int32), 0, NX - 1)
    iy = jnp.clip(((gy - YB[0]) * (1.0 / YB[2])).astype(jnp.int32), 0, NY - 1)
    flat = ix * NY + iy
    idx_ref[...] = jnp.where(keep, flat, -1)


def _compute_idx(gt):
    return pl.pallas_call(
        _idx_body,
        out_shape=jax.ShapeDtypeStruct((BD, HW), jnp.int32),
    )(gt)


def _sc_scatter(x4, idx3):
    mesh = plsc.VectorSubcoreMesh(core_axis_name="c", subcore_axis_name="s")
    cp = pltpu.CompilerParams()
    if "needs_layout_passes" in pltpu.CompilerParams.__dataclass_fields__:
        cp = dataclasses.replace(cp, needs_layout_passes=False)

    @functools.partial(
        pl.kernel,
        compiler_params=cp,
        out_type=jax.ShapeDtypeStruct((B, C, NCELL), jnp.float32),
        mesh=mesh,
        scratch_types=(
            [pltpu.VMEM((NCELL,), jnp.float32) for _ in range(CH_PER_TILE)]
            + [
                pltpu.VMEM((2, CH_PER_TILE, HW), jnp.float32),
                pltpu.VMEM((2, HW), jnp.int32),
                pltpu.SemaphoreType.DMA,
                pltpu.SemaphoreType.DMA,
                pltpu.SemaphoreType.DMA,
                pltpu.SemaphoreType.DMA,
            ]
        ),
    )
    def sc_kernel(x_hbm, idx_hbm, out_hbm, g0, g1, g2, g3, xbuf, ibuf,
                  sx0, sx1, si0, si1):
        grids = [g0, g1, g2, g3]
        sx = [sx0, sx1]
        si = [si0, si1]
        wid = lax.axis_index("s") * 2 + lax.axis_index("c")
        cgrp = lax.rem(wid, TILES_PER_BATCH)
        bsel = wid // TILES_PER_BATCH
        c0 = cgrp * CH_PER_TILE

        zero16 = jnp.zeros((LANES,), jnp.float32)

        def start_d(b, d, s):
            pltpu.make_async_copy(
                x_hbm.at[b, d, pl.ds(c0, CH_PER_TILE)], xbuf.at[s], sx[s]
            ).start()
            pltpu.make_async_copy(idx_hbm.at[b, d], ibuf.at[s], si[s]).start()

        def wait_d(s):
            pltpu.make_async_copy(
                x_hbm.at[0, 0, pl.ds(0, CH_PER_TILE)], xbuf.at[s], sx[s]
            ).wait()
            pltpu.make_async_copy(idx_hbm.at[0, 0], ibuf.at[s], si[s]).wait()

        def compute(s):
            @pl.loop(0, GROUPS, unroll=4)
            def _per_group(g):
                base = g * LANES
                idxv = ibuf[s, pl.ds(base, LANES)]
                mask = idxv >= 0
                for c in range(CH_PER_TILE):
                    vals = xbuf[s, c, pl.ds(base, LANES)]
                    plsc.addupdate_scatter(grids[c], [idxv], vals, mask=mask)

        for p in range(NUM_PASSES):
            b = p * BATCHES_PER_PASS + bsel

            start_d(b, 0, 0)

            @pl.loop(0, NCELL // LANES, unroll=8)
            def _zero(i):
                for c in range(CH_PER_TILE):
                    grids[c][pl.ds(i * LANES, LANES)] = zero16

            @pl.loop(0, (D - 1) // 2)
            def _per_pair(t):
                d = t * 2
                start_d(b, d + 1, 1)
                wait_d(0)
                compute(0)
                start_d(b, d + 2, 0)
                wait_d(1)
                compute(1)

            wait_d(0)
            compute(0)

            for c in range(CH_PER_TILE):
                pltpu.sync_copy(grids[c], out_hbm.at[b, c0 + c])

    return sc_kernel(x4, idx3)


@jax.jit
def kernel(geom_feats, x):
    gt = jnp.transpose(geom_feats.reshape(BD, HW, 3), (2, 0, 1))
    x4 = x.reshape(B, D, C, HW)
    idx = _compute_idx(gt)
    out = _sc_scatter(x4, idx.reshape(B, D, HW))
    return out.reshape(B, C, NX, NY)
